# baseline (speedup 1.0000x reference)
import functools

import jax
import jax.numpy as jnp
from jax import lax
from jax.experimental import pallas as pl
from jax.experimental.pallas import tpu as pltpu
from jax.experimental.pallas import tpu_sc as plsc

_B = 4
_N = 20000
_NPAD = 20480
_ROWS = _NPAD // 128
_PRE = 2048
_POST = 500
_THRESH = 0.7
_SLOTS = 512
_TILE = 256
_F = 16
_OUTROWS = _PRE + 8
_NEG = -1e30


def _select_body(s_ref, pos_ref):
    s = s_ref[0]
    key = lax.bitcast_convert_type(s, jnp.int32)
    skey = key ^ (lax.shift_right_arithmetic(key, 31) & jnp.int32(0x7FFFFFFF))

    def bit_step(i, t):
        cand = t + lax.shift_left(jnp.int32(1), 31 - i)
        cnt = jnp.sum((skey >= cand).astype(jnp.int32))
        return jnp.where(cnt >= _PRE, cand, t)

    t = lax.fori_loop(0, 32, bit_step, jnp.int32(-2147483648))

    gt = skey > t
    eq = skey == t
    c1 = jnp.sum(gt.astype(jnp.int32))
    need = (_PRE - c1).astype(jnp.float32)

    u_row = jax.lax.broadcasted_iota(jnp.int32, (128, 128), 0)
    u_col = jax.lax.broadcasted_iota(jnp.int32, (128, 128), 1)
    u128 = jnp.where(u_row < u_col, 1.0, 0.0).astype(jnp.bfloat16)
    l_row = jax.lax.broadcasted_iota(jnp.int32, (_ROWS, _ROWS), 0)
    l_col = jax.lax.broadcasted_iota(jnp.int32, (_ROWS, _ROWS), 1)
    l160 = jnp.where(l_col < l_row, 1.0, 0.0).astype(jnp.bfloat16)

    def ex_prefix(m):
        within = lax.dot_general(m, u128, (((1,), (0,)), ((), ())),
                                 preferred_element_type=jnp.float32)
        rs = jnp.sum(m.astype(jnp.float32), axis=1, keepdims=True)
        roff = lax.dot_general(l160, rs.astype(jnp.bfloat16),
                               (((1,), (0,)), ((), ())),
                               preferred_element_type=jnp.float32)
        return within + roff

    tie_rank = ex_prefix(eq.astype(jnp.bfloat16))
    selected = gt | (eq & (tie_rank < need))
    p = ex_prefix(selected.astype(jnp.bfloat16))
    pos_ref[0] = jnp.where(selected, p, float(_PRE)).astype(jnp.int32)


_SC_CORES = 2
_SC_SUBCORES = 16
_NTILES = _SC_CORES * _SC_SUBCORES
_CHUNK = _NPAD // _NTILES
_KSUB = _CHUNK // 128


@functools.cache
def _get_sc_scatter():
    mesh = plsc.VectorSubcoreMesh(core_axis_name="c", subcore_axis_name="s")

    @functools.partial(
        pl.kernel,
        mesh=mesh,
        out_type=jax.ShapeDtypeStruct((_B, _OUTROWS, _F), jnp.float32),
        scratch_types=[
            pltpu.VMEM((_KSUB, 128), jnp.int32),
            pltpu.VMEM((_CHUNK, _F), jnp.float32),
            pltpu.SemaphoreType.DMA,
        ],
        compiler_params=pltpu.CompilerParams(use_tc_tiling_on_sc=False),
    )
    def _sc_scatter(table_hbm, pos_hbm, out_hbm, pos_v, rows_v, sem):
        wid = lax.axis_index("s") * _SC_CORES + lax.axis_index("c")
        base = wid * _CHUNK
        for b in range(_B):
            pltpu.sync_copy(pos_hbm.at[wid, b], pos_v)
            pltpu.sync_copy(table_hbm.at[b, pl.ds(base, _CHUNK)], rows_v)
            for k in range(_KSUB):
                pltpu.async_copy(
                    rows_v.at[pl.ds(k * 128, 128)],
                    out_hbm.at[b].at[pos_v.at[k]],
                    sem).wait()

    return _sc_scatter


def _nms_body(feat_ref, featT_ref, out_ref, adj_ref, rank_ref):
    xr = featT_ref[0, 0:1, :]
    yr = featT_ref[0, 1:2, :]
    dxr = featT_ref[0, 3:4, :]
    dyr = featT_ref[0, 4:5, :]
    scr = featT_ref[0, 7:8, :]
    idr = featT_ref[0, 9:10, :]
    x1r = xr - dxr * 0.5
    x2r = xr + dxr * 0.5
    y1r = yr - dyr * 0.5
    y2r = yr + dyr * 0.5
    ar = dxr * dyr

    def build(ti, carry):
        ft = feat_ref[0, pl.ds(ti * _TILE, _TILE), :]
        xc = ft[:, 0:1]
        yc = ft[:, 1:2]
        dxc = ft[:, 3:4]
        dyc = ft[:, 4:5]
        scc = ft[:, 7:8]
        idc = ft[:, 9:10]
        x1c = xc - dxc * 0.5
        x2c = xc + dxc * 0.5
        y1c = yc - dyc * 0.5
        y2c = yc + dyc * 0.5
        ac = dxc * dyc
        ix = jnp.clip(jnp.minimum(x2c, x2r) - jnp.maximum(x1c, x1r), 0.0)
        iy = jnp.clip(jnp.minimum(y2c, y2r) - jnp.maximum(y1c, y1r), 0.0)
        inter = ix * iy
        union = ac + ar - inter
        iou = inter / jnp.maximum(union, 1e-6)
        before = (scc > scr) | ((scc == scr) & (idc < idr))
        adj_ref[pl.ds(ti * _TILE, _TILE), :] = jnp.where(
            (iou > _THRESH) & before, 1.0, 0.0).astype(jnp.bfloat16)
        rank_ref[pl.ds(ti * _TILE, _TILE), :] = jnp.where(
            before, 1.0, 0.0).astype(jnp.bfloat16)
        return carry

    lax.fori_loop(0, _PRE // _TILE, build, 0)

    def cond(c):
        return c[1]

    def body(c):
        keep, _ = c
        supp = lax.dot_general(
            keep, adj_ref[...], (((1,), (0,)), ((), ())),
            preferred_element_type=jnp.float32)
        new = (supp == 0.0).astype(jnp.bfloat16)
        changed = jnp.sum(jnp.abs(new.astype(jnp.float32)
                                  - keep.astype(jnp.float32))) > 0.0
        return new, changed

    keep0 = jnp.ones((1, _PRE), jnp.bfloat16)
    keep, _ = lax.while_loop(cond, body, (keep0, jnp.bool_(True)))

    p = lax.dot_general(
        keep, rank_ref[...], (((1,), (0,)), ((), ())),
        preferred_element_type=jnp.float32)
    keep32 = keep.astype(jnp.float32)

    srow = jax.lax.broadcasted_iota(jnp.int32, (_SLOTS, _PRE), 0).astype(jnp.float32)
    sel = jnp.where((p == srow) & (keep32 > 0.5), 1.0, 0.0)
    out_ref[0] = lax.dot_general(
        sel, feat_ref[0], (((1,), (0,)), ((), ())),
        precision=lax.Precision.HIGHEST,
        preferred_element_type=jnp.float32)


def kernel(batch_box_preds, batch_cls_preds):
    f32 = jnp.float32
    scores = jnp.max(batch_cls_preds, axis=-1)
    labp = (jnp.argmax(batch_cls_preds, axis=-1) + 1).astype(f32)
    idxf = jnp.broadcast_to(
        jnp.arange(_N, dtype=f32)[None, :], (_B, _N))

    table = jnp.concatenate(
        [batch_box_preds, scores[..., None], labp[..., None],
         idxf[..., None], jnp.zeros((_B, _N, _F - 10), f32)], axis=-1)
    table = jnp.pad(table, ((0, 0), (0, _NPAD - _N), (0, 0)))

    spad = jnp.pad(scores, ((0, 0), (0, _NPAD - _N)),
                   constant_values=_NEG).reshape(_B, _ROWS, 128)

    pos = pl.pallas_call(
        _select_body,
        grid=(_B,),
        in_specs=[pl.BlockSpec((1, _ROWS, 128), lambda i: (i, 0, 0))],
        out_specs=pl.BlockSpec((1, _ROWS, 128), lambda i: (i, 0, 0)),
        out_shape=jax.ShapeDtypeStruct((_B, _ROWS, 128), jnp.int32),
        compiler_params=pltpu.CompilerParams(
            dimension_semantics=("parallel",)),
    )(spad)

    pos_sc = jnp.transpose(
        pos.reshape(_B, _NTILES, _KSUB, 128), (1, 0, 2, 3))

    gathered = _get_sc_scatter()(table, pos_sc)

    feat = gathered[:, :_PRE, :]
    featT = jnp.transpose(feat, (0, 2, 1))

    out = pl.pallas_call(
        _nms_body,
        grid=(_B,),
        in_specs=[
            pl.BlockSpec((1, _PRE, _F), lambda i: (i, 0, 0)),
            pl.BlockSpec((1, _F, _PRE), lambda i: (i, 0, 0)),
        ],
        out_specs=pl.BlockSpec((1, _SLOTS, _F), lambda i: (i, 0, 0)),
        out_shape=jax.ShapeDtypeStruct((_B, _SLOTS, _F), jnp.float32),
        compiler_params=pltpu.CompilerParams(
            dimension_semantics=("parallel",)),
        scratch_shapes=[
            pltpu.VMEM((_PRE, _PRE), jnp.bfloat16),
            pltpu.VMEM((_PRE, _PRE), jnp.bfloat16),
        ],
    )(feat, featT)

    rois = out[:, :_POST, 0:7]
    roi_scores = out[:, :_POST, 7]
    roi_labels = jnp.round(out[:, :_POST, 8]).astype(jnp.int32)
    return rois, roi_scores, roi_labels


# speedup vs baseline: 1.0106x; 1.0106x over previous
import functools

import jax
import jax.numpy as jnp
from jax import lax
from jax.experimental import pallas as pl
from jax.experimental.pallas import tpu as pltpu
from jax.experimental.pallas import tpu_sc as plsc

_B = 4
_N = 20000
_NPAD = 20480
_ROWS = _NPAD // 128
_PRE = 2048
_POST = 500
_THRESH = 0.7
_SLOTS = 512
_TILE = 256
_F = 16
_OUTROWS = _PRE + 8
_NEG = -1e30


def _select_body(s_ref, pos_ref):
    s = s_ref[0]
    key = lax.bitcast_convert_type(s, jnp.int32)
    skey = key ^ (lax.shift_right_arithmetic(key, 31) & jnp.int32(0x7FFFFFFF))

    def bit_step(i, t):
        cand = t + lax.shift_left(jnp.int32(1), 31 - i)
        cnt = jnp.sum((skey >= cand).astype(jnp.int32))
        return jnp.where(cnt >= _PRE, cand, t)

    t = lax.fori_loop(0, 32, bit_step, jnp.int32(-2147483648))

    gt = skey > t
    eq = skey == t
    c1 = jnp.sum(gt.astype(jnp.int32))
    need = (_PRE - c1).astype(jnp.float32)

    u_row = jax.lax.broadcasted_iota(jnp.int32, (128, 128), 0)
    u_col = jax.lax.broadcasted_iota(jnp.int32, (128, 128), 1)
    u128 = jnp.where(u_row < u_col, 1.0, 0.0).astype(jnp.bfloat16)
    l_row = jax.lax.broadcasted_iota(jnp.int32, (_ROWS, _ROWS), 0)
    l_col = jax.lax.broadcasted_iota(jnp.int32, (_ROWS, _ROWS), 1)
    l160 = jnp.where(l_col < l_row, 1.0, 0.0).astype(jnp.bfloat16)

    def ex_prefix(m):
        within = lax.dot_general(m, u128, (((1,), (0,)), ((), ())),
                                 preferred_element_type=jnp.float32)
        rs = jnp.sum(m.astype(jnp.float32), axis=1, keepdims=True)
        roff = lax.dot_general(l160, rs.astype(jnp.bfloat16),
                               (((1,), (0,)), ((), ())),
                               preferred_element_type=jnp.float32)
        return within + roff

    tie_rank = ex_prefix(eq.astype(jnp.bfloat16))
    selected = gt | (eq & (tie_rank < need))
    p = ex_prefix(selected.astype(jnp.bfloat16))
    pos_ref[0] = jnp.where(selected, p, float(_PRE)).astype(jnp.int32)


_SC_CORES = 2
_SC_SUBCORES = 16
_NTILES = _SC_CORES * _SC_SUBCORES
_CHUNK = _NPAD // _NTILES
_KSUB = _CHUNK // 128


@functools.cache
def _get_sc_scatter():
    mesh = plsc.VectorSubcoreMesh(core_axis_name="c", subcore_axis_name="s")

    @functools.partial(
        pl.kernel,
        mesh=mesh,
        out_type=jax.ShapeDtypeStruct((_B, _OUTROWS, _F), jnp.float32),
        scratch_types=[
            pltpu.VMEM((_B, _KSUB, 128), jnp.int32),
            pltpu.VMEM((_B, _CHUNK, _F), jnp.float32),
            pltpu.SemaphoreType.DMA,
            pltpu.SemaphoreType.DMA,
        ],
        compiler_params=pltpu.CompilerParams(use_tc_tiling_on_sc=False),
    )
    def _sc_scatter(table_hbm, pos_hbm, out_hbm, pos_v, rows_v, lsem, ssem):
        wid = lax.axis_index("s") * _SC_CORES + lax.axis_index("c")
        base = wid * _CHUNK
        loads = []
        for b in range(_B):
            loads.append(pltpu.async_copy(pos_hbm.at[wid, b], pos_v.at[b], lsem))
            loads.append(pltpu.async_copy(
                table_hbm.at[b, pl.ds(base, _CHUNK)], rows_v.at[b], lsem))
        for c in loads:
            c.wait()
        scats = []
        for b in range(_B):
            for k in range(_KSUB):
                scats.append(pltpu.async_copy(
                    rows_v.at[b, pl.ds(k * 128, 128)],
                    out_hbm.at[b].at[pos_v.at[b, k]],
                    ssem))
        for c in scats:
            c.wait()

    return _sc_scatter


def _nms_body(feat_ref, featT_ref, out_ref, adj_ref, rank_ref):
    xr = featT_ref[0, 0:1, :]
    yr = featT_ref[0, 1:2, :]
    dxr = featT_ref[0, 3:4, :]
    dyr = featT_ref[0, 4:5, :]
    scr = featT_ref[0, 7:8, :]
    idr = featT_ref[0, 9:10, :]
    x1r = xr - dxr * 0.5
    x2r = xr + dxr * 0.5
    y1r = yr - dyr * 0.5
    y2r = yr + dyr * 0.5
    ar = dxr * dyr

    def build(ti, carry):
        ft = feat_ref[0, pl.ds(ti * _TILE, _TILE), :]
        xc = ft[:, 0:1]
        yc = ft[:, 1:2]
        dxc = ft[:, 3:4]
        dyc = ft[:, 4:5]
        scc = ft[:, 7:8]
        idc = ft[:, 9:10]
        x1c = xc - dxc * 0.5
        x2c = xc + dxc * 0.5
        y1c = yc - dyc * 0.5
        y2c = yc + dyc * 0.5
        ac = dxc * dyc
        ix = jnp.clip(jnp.minimum(x2c, x2r) - jnp.maximum(x1c, x1r), 0.0)
        iy = jnp.clip(jnp.minimum(y2c, y2r) - jnp.maximum(y1c, y1r), 0.0)
        inter = ix * iy
        union = ac + ar - inter
        iou = inter / jnp.maximum(union, 1e-6)
        before = (scc > scr) | ((scc == scr) & (idc < idr))
        adj_ref[pl.ds(ti * _TILE, _TILE), :] = jnp.where(
            (iou > _THRESH) & before, 1.0, 0.0).astype(jnp.bfloat16)
        rank_ref[pl.ds(ti * _TILE, _TILE), :] = jnp.where(
            before, 1.0, 0.0).astype(jnp.bfloat16)
        return carry

    lax.fori_loop(0, _PRE // _TILE, build, 0)

    def cond(c):
        return c[1]

    def body(c):
        keep, _ = c
        supp = lax.dot_general(
            keep, adj_ref[...], (((1,), (0,)), ((), ())),
            preferred_element_type=jnp.float32)
        new = (supp == 0.0).astype(jnp.bfloat16)
        changed = jnp.sum(jnp.abs(new.astype(jnp.float32)
                                  - keep.astype(jnp.float32))) > 0.0
        return new, changed

    keep0 = jnp.ones((1, _PRE), jnp.bfloat16)
    keep, _ = lax.while_loop(cond, body, (keep0, jnp.bool_(True)))

    p = lax.dot_general(
        keep, rank_ref[...], (((1,), (0,)), ((), ())),
        preferred_element_type=jnp.float32)
    keep32 = keep.astype(jnp.float32)

    srow = jax.lax.broadcasted_iota(jnp.int32, (_SLOTS, _PRE), 0).astype(jnp.float32)
    sel = jnp.where((p == srow) & (keep32 > 0.5), 1.0, 0.0)
    out_ref[0] = lax.dot_general(
        sel, feat_ref[0], (((1,), (0,)), ((), ())),
        precision=lax.Precision.HIGHEST,
        preferred_element_type=jnp.float32)


def kernel(batch_box_preds, batch_cls_preds):
    f32 = jnp.float32
    scores = jnp.max(batch_cls_preds, axis=-1)
    labp = (jnp.argmax(batch_cls_preds, axis=-1) + 1).astype(f32)
    idxf = jnp.broadcast_to(
        jnp.arange(_N, dtype=f32)[None, :], (_B, _N))

    table = jnp.concatenate(
        [batch_box_preds, scores[..., None], labp[..., None],
         idxf[..., None], jnp.zeros((_B, _N, _F - 10), f32)], axis=-1)
    table = jnp.pad(table, ((0, 0), (0, _NPAD - _N), (0, 0)))

    spad = jnp.pad(scores, ((0, 0), (0, _NPAD - _N)),
                   constant_values=_NEG).reshape(_B, _ROWS, 128)

    pos = pl.pallas_call(
        _select_body,
        grid=(_B,),
        in_specs=[pl.BlockSpec((1, _ROWS, 128), lambda i: (i, 0, 0))],
        out_specs=pl.BlockSpec((1, _ROWS, 128), lambda i: (i, 0, 0)),
        out_shape=jax.ShapeDtypeStruct((_B, _ROWS, 128), jnp.int32),
        compiler_params=pltpu.CompilerParams(
            dimension_semantics=("parallel",)),
    )(spad)

    pos_sc = jnp.transpose(
        pos.reshape(_B, _NTILES, _KSUB, 128), (1, 0, 2, 3))

    gathered = _get_sc_scatter()(table, pos_sc)

    feat = gathered[:, :_PRE, :]
    featT = jnp.transpose(feat, (0, 2, 1))

    out = pl.pallas_call(
        _nms_body,
        grid=(_B,),
        in_specs=[
            pl.BlockSpec((1, _PRE, _F), lambda i: (i, 0, 0)),
            pl.BlockSpec((1, _F, _PRE), lambda i: (i, 0, 0)),
        ],
        out_specs=pl.BlockSpec((1, _SLOTS, _F), lambda i: (i, 0, 0)),
        out_shape=jax.ShapeDtypeStruct((_B, _SLOTS, _F), jnp.float32),
        compiler_params=pltpu.CompilerParams(
            dimension_semantics=("parallel",)),
        scratch_shapes=[
            pltpu.VMEM((_PRE, _PRE), jnp.bfloat16),
            pltpu.VMEM((_PRE, _PRE), jnp.bfloat16),
        ],
    )(feat, featT)

    rois = out[:, :_POST, 0:7]
    roi_scores = out[:, :_POST, 7]
    roi_labels = jnp.round(out[:, :_POST, 8]).astype(jnp.int32)
    return rois, roi_scores, roi_labels


# speedup vs baseline: 1.5149x; 1.4991x over previous
import functools

import jax
import jax.numpy as jnp
from jax import lax
from jax.experimental import pallas as pl
from jax.experimental.pallas import tpu as pltpu
from jax.experimental.pallas import tpu_sc as plsc

_B = 4
_N = 20000
_NPAD = 20480
_ROWS = _NPAD // 128
_PRE = 2048
_POST = 500
_THRESH = 0.7
_SLOTS = 512
_TILE = 256
_F = 16
_OUTROWS = _PRE + 8
_NEG = -1e30


def _select_body(s_ref, pos_ref):
    s = s_ref[0]
    key = lax.bitcast_convert_type(s, jnp.int32)
    skey = key ^ (lax.shift_right_arithmetic(key, 31) & jnp.int32(0x7FFFFFFF))

    def bit_step(i, t):
        cand = t + lax.shift_left(jnp.int32(1), 31 - i)
        cnt = jnp.sum((skey >= cand).astype(jnp.int32))
        return jnp.where(cnt >= _PRE, cand, t)

    t = lax.fori_loop(0, 32, bit_step, jnp.int32(-2147483648))

    gt = skey > t
    eq = skey == t
    c1 = jnp.sum(gt.astype(jnp.int32))
    need = (_PRE - c1).astype(jnp.float32)

    u_row = jax.lax.broadcasted_iota(jnp.int32, (128, 128), 0)
    u_col = jax.lax.broadcasted_iota(jnp.int32, (128, 128), 1)
    u128 = jnp.where(u_row < u_col, 1.0, 0.0).astype(jnp.bfloat16)
    l_row = jax.lax.broadcasted_iota(jnp.int32, (_ROWS, _ROWS), 0)
    l_col = jax.lax.broadcasted_iota(jnp.int32, (_ROWS, _ROWS), 1)
    l160 = jnp.where(l_col < l_row, 1.0, 0.0).astype(jnp.bfloat16)

    def ex_prefix(m):
        within = lax.dot_general(m, u128, (((1,), (0,)), ((), ())),
                                 preferred_element_type=jnp.float32)
        rs = jnp.sum(m.astype(jnp.float32), axis=1, keepdims=True)
        roff = lax.dot_general(l160, rs.astype(jnp.bfloat16),
                               (((1,), (0,)), ((), ())),
                               preferred_element_type=jnp.float32)
        return within + roff

    tie_rank = ex_prefix(eq.astype(jnp.bfloat16))
    selected = gt | (eq & (tie_rank < need))
    p = ex_prefix(selected.astype(jnp.bfloat16))
    pos_ref[0] = jnp.where(selected, p, float(_PRE)).astype(jnp.int32)


_SC_CORES = 2
_SC_SUBCORES = 16
_NTILES = _SC_CORES * _SC_SUBCORES
_CHUNK = _NPAD // _NTILES
_KSUB = _CHUNK // 128


_BPC = _B // _SC_CORES
_EPT = _NPAD // _SC_SUBCORES
_VPT = _EPT // 16
_SPT = _PRE // _SC_SUBCORES


@functools.cache
def _get_sc_compact():
    mesh = plsc.VectorSubcoreMesh(core_axis_name="c", subcore_axis_name="s")

    @functools.partial(
        pl.kernel,
        mesh=mesh,
        out_type=jax.ShapeDtypeStruct((_B, _PRE, _F), jnp.float32),
        scratch_types=[
            pltpu.VMEM((_BPC, _VPT, 16), jnp.int32),
            pltpu.VMEM((_BPC, _PRE), jnp.int32),
            pltpu.VMEM((_BPC, _SC_SUBCORES, _SPT), jnp.int32),
            pltpu.VMEM((_BPC, _SPT), jnp.int32),
            pltpu.VMEM((_BPC, _SPT, _F), jnp.float32),
            pltpu.VMEM_SHARED((_BPC, _SC_SUBCORES, _PRE), jnp.int32),
            pltpu.SemaphoreType.DMA,
            pltpu.SemaphoreType.DMA,
            pltpu.SemaphoreType.DMA,
        ],
        compiler_params=pltpu.CompilerParams(
            needs_layout_passes=False, use_tc_tiling_on_sc=False),
    )
    def _sc_compact(table_hbm, pos_hbm, out_hbm,
                    pos_v, acc_v, comb_v, gidx_v, grows_v, spm,
                    lsem, psem, gsem):
        cid = lax.axis_index("c")
        sid = lax.axis_index("s")
        base = sid * _EPT
        lane = lax.iota(jnp.int32, 16)

        loads = [pltpu.async_copy(pos_hbm.at[cid * _BPC + bb, sid],
                                  pos_v.at[bb], lsem)
                 for bb in range(_BPC)]
        for c in loads:
            c.wait()

        pubs = []
        for bb in range(_BPC):
            def zero(i, _):
                acc_v[bb, pl.ds(i * 16, 16)] = jnp.zeros((16,), jnp.int32)
                return _
            lax.fori_loop(0, _PRE // 16, zero, 0)

            def scat(j, _):
                pv = pos_v[bb, j]
                vals = base + j * 16 + lane + 1
                plsc.store_scatter(acc_v.at[bb], [pv], vals, mask=pv < _PRE)
                return _
            lax.fori_loop(0, _VPT, scat, 0)
            pubs.append(pltpu.async_copy(acc_v.at[bb], spm.at[bb, sid], psem))
        for c in pubs:
            c.wait()
        plsc.subcore_barrier()

        for bb in range(_BPC):
            pltpu.sync_copy(spm.at[bb, :, pl.ds(sid * _SPT, _SPT)],
                            comb_v.at[bb])
        gats = []
        for bb in range(_BPC):
            for j in range(_SPT // 16):
                m = comb_v[bb, 0, pl.ds(j * 16, 16)]
                for r in range(1, _SC_SUBCORES):
                    m = jnp.maximum(m, comb_v[bb, r, pl.ds(j * 16, 16)])
                gidx_v[bb, pl.ds(j * 16, 16)] = m - 1
            gats.append(pltpu.async_copy(
                table_hbm.at[cid * _BPC + bb].at[gidx_v.at[bb]],
                grows_v.at[bb], gsem))
        for c in gats:
            c.wait()
        for bb in range(_BPC):
            pltpu.sync_copy(grows_v.at[bb],
                            out_hbm.at[cid * _BPC + bb, pl.ds(sid * _SPT, _SPT)])

    return _sc_compact


def _nms_body(feat_ref, featT_ref, out_ref, adj_ref, rank_ref):
    xr = featT_ref[0, 0:1, :]
    yr = featT_ref[0, 1:2, :]
    dxr = featT_ref[0, 3:4, :]
    dyr = featT_ref[0, 4:5, :]
    scr = featT_ref[0, 7:8, :]
    idr = featT_ref[0, 9:10, :]
    x1r = xr - dxr * 0.5
    x2r = xr + dxr * 0.5
    y1r = yr - dyr * 0.5
    y2r = yr + dyr * 0.5
    ar = dxr * dyr

    def build(ti, carry):
        ft = feat_ref[0, pl.ds(ti * _TILE, _TILE), :]
        xc = ft[:, 0:1]
        yc = ft[:, 1:2]
        dxc = ft[:, 3:4]
        dyc = ft[:, 4:5]
        scc = ft[:, 7:8]
        idc = ft[:, 9:10]
        x1c = xc - dxc * 0.5
        x2c = xc + dxc * 0.5
        y1c = yc - dyc * 0.5
        y2c = yc + dyc * 0.5
        ac = dxc * dyc
        ix = jnp.clip(jnp.minimum(x2c, x2r) - jnp.maximum(x1c, x1r), 0.0)
        iy = jnp.clip(jnp.minimum(y2c, y2r) - jnp.maximum(y1c, y1r), 0.0)
        inter = ix * iy
        union = ac + ar - inter
        iou = inter / jnp.maximum(union, 1e-6)
        before = (scc > scr) | ((scc == scr) & (idc < idr))
        adj_ref[pl.ds(ti * _TILE, _TILE), :] = jnp.where(
            (iou > _THRESH) & before, 1.0, 0.0).astype(jnp.bfloat16)
        rank_ref[pl.ds(ti * _TILE, _TILE), :] = jnp.where(
            before, 1.0, 0.0).astype(jnp.bfloat16)
        return carry

    lax.fori_loop(0, _PRE // _TILE, build, 0)

    def cond(c):
        return c[1]

    def body(c):
        keep, _ = c
        supp = lax.dot_general(
            keep, adj_ref[...], (((1,), (0,)), ((), ())),
            preferred_element_type=jnp.float32)
        new = (supp == 0.0).astype(jnp.bfloat16)
        changed = jnp.sum(jnp.abs(new.astype(jnp.float32)
                                  - keep.astype(jnp.float32))) > 0.0
        return new, changed

    keep0 = jnp.ones((1, _PRE), jnp.bfloat16)
    keep, _ = lax.while_loop(cond, body, (keep0, jnp.bool_(True)))

    p = lax.dot_general(
        keep, rank_ref[...], (((1,), (0,)), ((), ())),
        preferred_element_type=jnp.float32)
    keep32 = keep.astype(jnp.float32)

    srow = jax.lax.broadcasted_iota(jnp.int32, (_SLOTS, _PRE), 0).astype(jnp.float32)
    sel = jnp.where((p == srow) & (keep32 > 0.5), 1.0, 0.0)
    out_ref[0] = lax.dot_general(
        sel, feat_ref[0], (((1,), (0,)), ((), ())),
        precision=lax.Precision.HIGHEST,
        preferred_element_type=jnp.float32)


def kernel(batch_box_preds, batch_cls_preds):
    f32 = jnp.float32
    scores = jnp.max(batch_cls_preds, axis=-1)
    labp = (jnp.argmax(batch_cls_preds, axis=-1) + 1).astype(f32)
    idxf = jnp.broadcast_to(
        jnp.arange(_N, dtype=f32)[None, :], (_B, _N))

    table = jnp.concatenate(
        [batch_box_preds, scores[..., None], labp[..., None],
         idxf[..., None], jnp.zeros((_B, _N, _F - 10), f32)], axis=-1)
    table = jnp.pad(table, ((0, 0), (0, _NPAD - _N), (0, 0)))

    spad = jnp.pad(scores, ((0, 0), (0, _NPAD - _N)),
                   constant_values=_NEG).reshape(_B, _ROWS, 128)

    pos = pl.pallas_call(
        _select_body,
        grid=(_B,),
        in_specs=[pl.BlockSpec((1, _ROWS, 128), lambda i: (i, 0, 0))],
        out_specs=pl.BlockSpec((1, _ROWS, 128), lambda i: (i, 0, 0)),
        out_shape=jax.ShapeDtypeStruct((_B, _ROWS, 128), jnp.int32),
        compiler_params=pltpu.CompilerParams(
            dimension_semantics=("parallel",)),
    )(spad)

    pos_sc = pos.reshape(_B, _SC_SUBCORES, _VPT, 16)

    feat = _get_sc_compact()(table, pos_sc)
    featT = jnp.transpose(feat, (0, 2, 1))

    out = pl.pallas_call(
        _nms_body,
        grid=(_B,),
        in_specs=[
            pl.BlockSpec((1, _PRE, _F), lambda i: (i, 0, 0)),
            pl.BlockSpec((1, _F, _PRE), lambda i: (i, 0, 0)),
        ],
        out_specs=pl.BlockSpec((1, _SLOTS, _F), lambda i: (i, 0, 0)),
        out_shape=jax.ShapeDtypeStruct((_B, _SLOTS, _F), jnp.float32),
        compiler_params=pltpu.CompilerParams(
            dimension_semantics=("parallel",)),
        scratch_shapes=[
            pltpu.VMEM((_PRE, _PRE), jnp.bfloat16),
            pltpu.VMEM((_PRE, _PRE), jnp.bfloat16),
        ],
    )(feat, featT)

    rois = out[:, :_POST, 0:7]
    roi_scores = out[:, :_POST, 7]
    roi_labels = jnp.round(out[:, :_POST, 8]).astype(jnp.int32)
    return rois, roi_scores, roi_labels
